# baseline (device time: 16736 ns/iter reference)
import jax
import jax.numpy as jnp
from jax import lax
from jax.experimental import pallas as pl
from jax.experimental.pallas import tpu as pltpu

K = 4


def kernel(x, pi):
    m, h, w = x.shape
    half = h // 2
    ch = half // K

    def body(pi_ref, x_ref, out_ref, ysend, yrecv, xsend, xrecv):
        my_x = lax.axis_index("x")
        my_y = lax.axis_index("y")
        my_z = lax.axis_index("z")
        dst_y = pi_ref[my_y]
        other_x = 1 - my_x

        barrier_sem = pltpu.get_barrier_semaphore()
        pl.semaphore_signal(
            barrier_sem,
            inc=1,
            device_id=(my_x, dst_y, my_z),
            device_id_type=pl.DeviceIdType.MESH,
        )
        pl.semaphore_signal(
            barrier_sem,
            inc=1,
            device_id=(other_x, my_y, my_z),
            device_id_type=pl.DeviceIdType.MESH,
        )
        pl.semaphore_wait(barrier_sem, 2)

        base = my_x * half
        y_rdmas = []
        for k in range(K):
            r = pl.ds(base + k * ch, ch)
            rd = pltpu.make_async_remote_copy(
                src_ref=x_ref.at[:, r],
                dst_ref=out_ref.at[:, r],
                send_sem=ysend.at[k],
                recv_sem=yrecv.at[k],
                device_id=(my_x, dst_y, my_z),
                device_id_type=pl.DeviceIdType.MESH,
            )
            rd.start()
            y_rdmas.append(rd)

        x_rdmas = []
        for k in range(K):
            y_rdmas[k].wait_recv()
            r = pl.ds(base + k * ch, ch)
            rd = pltpu.make_async_remote_copy(
                src_ref=out_ref.at[:, r],
                dst_ref=out_ref.at[:, r],
                send_sem=xsend.at[k],
                recv_sem=xrecv.at[k],
                device_id=(other_x, my_y, my_z),
                device_id_type=pl.DeviceIdType.MESH,
            )
            rd.start()
            x_rdmas.append(rd)

        for k in range(K):
            x_rdmas[k].wait_recv()
        for k in range(K):
            y_rdmas[k].wait_send()
            x_rdmas[k].wait_send()

    return pl.pallas_call(
        body,
        out_shape=jax.ShapeDtypeStruct((m, h, w), jnp.float32),
        in_specs=[
            pl.BlockSpec(memory_space=pltpu.SMEM),
            pl.BlockSpec(memory_space=pltpu.VMEM),
        ],
        out_specs=pl.BlockSpec(memory_space=pltpu.VMEM),
        scratch_shapes=[
            pltpu.SemaphoreType.DMA((K,)),
            pltpu.SemaphoreType.DMA((K,)),
            pltpu.SemaphoreType.DMA((K,)),
            pltpu.SemaphoreType.DMA((K,)),
        ],
        compiler_params=pltpu.CompilerParams(collective_id=0),
    )(pi, x)


# device time: 14079 ns/iter; 1.1887x vs baseline; 1.1887x over previous
import jax
import jax.numpy as jnp
from jax import lax
from jax.experimental import pallas as pl
from jax.experimental.pallas import tpu as pltpu


def kernel(x, pi):
    m, h, w = x.shape
    half = h // 2

    def body(pi_ref, x_ref, out_ref, ysend, yrecv, xsend, xrecv):
        my_x = lax.axis_index("x")
        my_y = lax.axis_index("y")
        my_z = lax.axis_index("z")
        dst_y = pi_ref[my_y]
        other_x = 1 - my_x

        barrier_sem = pltpu.get_barrier_semaphore()
        pl.semaphore_signal(
            barrier_sem,
            inc=1,
            device_id=(my_x, dst_y, my_z),
            device_id_type=pl.DeviceIdType.MESH,
        )
        pl.semaphore_signal(
            barrier_sem,
            inc=1,
            device_id=(other_x, my_y, my_z),
            device_id_type=pl.DeviceIdType.MESH,
        )
        pl.semaphore_wait(barrier_sem, 2)

        r_mine = pl.ds(my_x * half, half)
        r_other = pl.ds(other_x * half, half)

        ry = pltpu.make_async_remote_copy(
            src_ref=x_ref.at[:, r_mine],
            dst_ref=out_ref.at[:, r_mine],
            send_sem=ysend,
            recv_sem=yrecv,
            device_id=(my_x, dst_y, my_z),
            device_id_type=pl.DeviceIdType.MESH,
        )
        rx = pltpu.make_async_remote_copy(
            src_ref=x_ref.at[:, r_other],
            dst_ref=out_ref.at[:, r_other],
            send_sem=xsend,
            recv_sem=xrecv,
            device_id=(other_x, my_y, my_z),
            device_id_type=pl.DeviceIdType.MESH,
        )
        ry.start()
        rx.start()
        ry.wait()
        rx.wait()

    return pl.pallas_call(
        body,
        out_shape=jax.ShapeDtypeStruct((m, h, w), jnp.float32),
        in_specs=[
            pl.BlockSpec(memory_space=pltpu.SMEM),
            pl.BlockSpec(memory_space=pltpu.VMEM),
        ],
        out_specs=pl.BlockSpec(memory_space=pltpu.VMEM),
        scratch_shapes=[
            pltpu.SemaphoreType.DMA,
            pltpu.SemaphoreType.DMA,
            pltpu.SemaphoreType.DMA,
            pltpu.SemaphoreType.DMA,
        ],
        compiler_params=pltpu.CompilerParams(collective_id=0),
    )(pi, x)


# device time: 12391 ns/iter; 1.3507x vs baseline; 1.1362x over previous
import jax
import jax.numpy as jnp
from jax import lax
from jax.experimental import pallas as pl
from jax.experimental.pallas import tpu as pltpu

SPLITS = ((0, 176), (176, 168), (344, 168))


def kernel(x, pi):
    m, h, w = x.shape

    def body(pi_ref, x_ref, out_ref, *sems):
        my_x = lax.axis_index("x")
        my_y = lax.axis_index("y")
        my_z = lax.axis_index("z")
        dst_y = pi_ref[my_y]
        other_x = 1 - my_x
        other_z = 1 - my_z

        targets = [
            (my_x, dst_y, my_z),
            (other_x, my_y, my_z),
            (my_x, my_y, other_z),
        ]

        barrier_sem = pltpu.get_barrier_semaphore()
        for t in targets:
            pl.semaphore_signal(
                barrier_sem,
                inc=1,
                device_id=t,
                device_id_type=pl.DeviceIdType.MESH,
            )
        pl.semaphore_wait(barrier_sem, 3)

        rdmas = []
        for i, ((start, size), tgt) in enumerate(zip(SPLITS, targets)):
            r = pl.ds(start, size)
            rd = pltpu.make_async_remote_copy(
                src_ref=x_ref.at[:, r],
                dst_ref=out_ref.at[:, r],
                send_sem=sems[2 * i],
                recv_sem=sems[2 * i + 1],
                device_id=tgt,
                device_id_type=pl.DeviceIdType.MESH,
            )
            rd.start()
            rdmas.append(rd)
        for rd in rdmas:
            rd.wait()

    return pl.pallas_call(
        body,
        out_shape=jax.ShapeDtypeStruct((m, h, w), jnp.float32),
        in_specs=[
            pl.BlockSpec(memory_space=pltpu.SMEM),
            pl.BlockSpec(memory_space=pltpu.VMEM),
        ],
        out_specs=pl.BlockSpec(memory_space=pltpu.VMEM),
        scratch_shapes=[pltpu.SemaphoreType.DMA] * 6,
        compiler_params=pltpu.CompilerParams(collective_id=0),
    )(pi, x)
